# Initial kernel scaffold; baseline (speedup 1.0000x reference)
import jax
import jax.numpy as jnp
from jax.experimental import pallas as pl


def kernel(image_emb, keys, W, b):
    raise NotImplementedError("write your pallas kernel here")


# speedup vs baseline: 1.8746x; 1.8746x over previous
import functools

import jax
import jax.numpy as jnp
from jax.experimental import pallas as pl
from jax.experimental.pallas import tpu as pltpu

_TOPK = 3
_BIGF = 3.0e38
_BIGI = 2**31 - 1
_TQ = 256
_TK = 2048


def _topk3(d, gi):
    vals, idxs = [], []
    for j in range(_TOPK):
        m = jnp.min(d, axis=1, keepdims=True)
        sel = jnp.min(jnp.where(d == m, gi, _BIGI), axis=1, keepdims=True)
        vals.append(m)
        idxs.append(sel)
        if j < _TOPK - 1:
            d = jnp.where(gi == sel, _BIGF, d)
    return jnp.concatenate(vals, axis=1), jnp.concatenate(idxs, axis=1)


def _retr_kernel(nkeys, img_ref, keys_ref, wt_ref, b_ref, ksq_ref,
                 outv_ref, outi_ref, proj_ref, qsq_ref, rv_ref, ri_ref):
    ki = pl.program_id(1)

    @pl.when(ki == 0)
    def _project():
        p = jax.lax.dot_general(
            img_ref[...], wt_ref[...], (((1,), (0,)), ((), ())),
            preferred_element_type=jnp.float32) + b_ref[...]
        proj_ref[...] = p
        qsq_ref[...] = jnp.sum(p * p, axis=1, keepdims=True)

    p = proj_ref[...]
    kb = keys_ref[...]
    mm = jax.lax.dot_general(p, kb, (((1,), (1,)), ((), ())),
                             preferred_element_type=jnp.float32)
    d = (qsq_ref[...] + ksq_ref[...]) - 2.0 * mm

    gi = ki * _TK + jax.lax.broadcasted_iota(jnp.int32, d.shape, 1)
    d = jnp.where(gi < nkeys, d, _BIGF)

    tv, ti = _topk3(d, gi)

    @pl.when(ki == 0)
    def _init():
        rv_ref[...] = tv
        ri_ref[...] = ti

    @pl.when(ki != 0)
    def _merge():
        cv = jnp.concatenate([rv_ref[...], tv], axis=1)
        ci = jnp.concatenate([ri_ref[...], ti], axis=1)
        nv, ni = _topk3(cv, ci)
        rv_ref[...] = nv
        ri_ref[...] = ni

    outv_ref[...] = -rv_ref[...]
    outi_ref[...] = ri_ref[...]


def kernel(image_emb, keys, W, b):
    Q, Din = image_emb.shape
    K, D = keys.shape
    nq = Q // _TQ
    nk = (K + _TK - 1) // _TK
    wt = W.T
    b2 = b.reshape(1, D)
    ksq = jnp.sum(keys * keys, axis=1)[None, :]
    vals, idx = pl.pallas_call(
        functools.partial(_retr_kernel, K),
        grid=(nq, nk),
        in_specs=[
            pl.BlockSpec((_TQ, Din), lambda qi, ki: (qi, 0)),
            pl.BlockSpec((_TK, D), lambda qi, ki: (ki, 0)),
            pl.BlockSpec((Din, D), lambda qi, ki: (0, 0)),
            pl.BlockSpec((1, D), lambda qi, ki: (0, 0)),
            pl.BlockSpec((1, _TK), lambda qi, ki: (0, ki)),
        ],
        out_specs=[
            pl.BlockSpec((_TQ, _TOPK), lambda qi, ki: (qi, 0)),
            pl.BlockSpec((_TQ, _TOPK), lambda qi, ki: (qi, 0)),
        ],
        out_shape=[
            jax.ShapeDtypeStruct((Q, _TOPK), jnp.float32),
            jax.ShapeDtypeStruct((Q, _TOPK), jnp.int32),
        ],
        scratch_shapes=[
            pltpu.VMEM((_TQ, D), jnp.float32),
            pltpu.VMEM((_TQ, 1), jnp.float32),
            pltpu.VMEM((_TQ, _TOPK), jnp.float32),
            pltpu.VMEM((_TQ, _TOPK), jnp.int32),
        ],
    )(image_emb, keys, wt, b2, ksq)
    return vals, idx
